# baseline (speedup 1.0000x reference)
import jax
import jax.numpy as jnp
from jax import lax
from jax.experimental import pallas as pl
from jax.experimental.pallas import tpu as pltpu
from jax.experimental.pallas import tpu_sc as plsc

_G0 = 128
_G1 = 128
_E = _G0 + _G1
_K = 4
_NEG = float("-inf")
_NC = 2
_NS = 16
_NW = _NC * _NS
_L = 16


def _matmul_kernel(x_ref, w_ref, b_ref, st_ref):
    st_ref[...] = jax.lax.dot_general(
        w_ref[...], x_ref[...], (((1,), (1,)), ((), ())),
        preferred_element_type=jnp.float32,
    ) + b_ref[...]


def _insert4(vs, ps, c, pc):
    for k in range(_K):
        swap = c > vs[k]
        vs[k], c = jnp.where(swap, c, vs[k]), jnp.where(swap, vs[k], c)
        ps[k], pc = jnp.where(swap, pc, ps[k]), jnp.where(swap, ps[k], pc)


def _sc_route(st_hbm, ids_hbm, lg_hbm, wt_hbm, sc_v, ids_v, lg_v, wt_v):
    tpw = st_hbm.shape[1] // _NW
    wid = lax.axis_index("s") * _NC + lax.axis_index("c")
    base = wid * tpw
    pltpu.sync_copy(st_hbm.at[:, pl.ds(base, tpw)], sc_v)

    neg = jnp.full((_L,), _NEG, jnp.float32)
    zero = jnp.zeros((_L,), jnp.int32)

    def group(g, carry):
        col = pl.ds(g * _L, _L)

        def stage(e0, e1):
            def estep(e, st):
                vs, ps = list(st[:_K]), list(st[_K:])
                _insert4(vs, ps, sc_v[e, col],
                         jnp.broadcast_to(e - e0, (_L,)))
                return (*vs, *ps)
            return lax.fori_loop(
                e0, e1, estep, (neg, neg, neg, neg, zero, zero, zero, zero))

        s0 = stage(0, _G0)
        s1 = stage(_G0, _E)
        v0, i0 = s0[:_K], s0[_K:]
        v1, i1 = s1[:_K], s1[_K:]

        vs, ps = [neg] * _K, [zero] * _K
        for b in range(_K):
            for j in range(_K):
                _insert4(vs, ps, v0[b] + v1[j], i0[b] * _G1 + i1[j])

        es = [jnp.exp(v - vs[0]) for v in vs]
        den = es[0] + es[1] + es[2] + es[3]
        for t in range(_K):
            ids_v[t, col] = ps[t]
            lg_v[t, col] = vs[t]
            wt_v[t, col] = es[t] / den
        return carry

    lax.fori_loop(0, tpw // _L, group, 0)
    pltpu.sync_copy(ids_v, ids_hbm.at[:, pl.ds(base, tpw)])
    pltpu.sync_copy(lg_v, lg_hbm.at[:, pl.ds(base, tpw)])
    pltpu.sync_copy(wt_v, wt_hbm.at[:, pl.ds(base, tpw)])


def kernel(input, W, b):
    n, d = input.shape
    bt = 1024
    scores_t = pl.pallas_call(
        _matmul_kernel,
        grid=(n // bt,),
        in_specs=[
            pl.BlockSpec((bt, d), lambda i: (i, 0)),
            pl.BlockSpec((_E, d), lambda i: (0, 0)),
            pl.BlockSpec((_E, 1), lambda i: (0, 0)),
        ],
        out_specs=pl.BlockSpec((_E, bt), lambda i: (0, i)),
        out_shape=jax.ShapeDtypeStruct((_E, n), jnp.float32),
    )(input, W, b.reshape(_E, 1))

    tpw = n // _NW
    route = pl.kernel(
        _sc_route,
        mesh=plsc.VectorSubcoreMesh(core_axis_name="c", subcore_axis_name="s"),
        out_type=[
            jax.ShapeDtypeStruct((_K, n), jnp.int32),
            jax.ShapeDtypeStruct((_K, n), jnp.float32),
            jax.ShapeDtypeStruct((_K, n), jnp.float32),
        ],
        scratch_types=[
            pltpu.VMEM((_E, tpw), jnp.float32),
            pltpu.VMEM((_K, tpw), jnp.int32),
            pltpu.VMEM((_K, tpw), jnp.float32),
            pltpu.VMEM((_K, tpw), jnp.float32),
        ],
    )
    ids_t, lg_t, wt_t = route(scores_t)
    return ids_t.T, lg_t.T, wt_t.T


kernel = jax.jit(kernel)
